# baseline (speedup 1.0000x reference)
import jax
import jax.numpy as jnp
from jax.experimental import pallas as pl
from jax.experimental.pallas import tpu as pltpu

_VMEM_LIMIT = 96 * 1024 * 1024


def _fused_decoder_kernel(x_ref, skip_ref, wup_ref, bup_ref, w1_ref, s1_ref,
                          w2_ref, s2_ref, o_ref):
    H, W, Cin = 32, 32, 256
    C = 128

    xc = x_ref[0].reshape(Cin, H * W).astype(jnp.bfloat16)
    b = bup_ref[...]
    taps = []
    for k in range(4):
        r = jax.lax.dot_general(
            xc, wup_ref[k], (((0,), (0,)), ((), ())),
            preferred_element_type=jnp.float32) + b
        taps.append(r.reshape(H, W, C).astype(jnp.bfloat16))
    row_even = jnp.stack([taps[0], taps[1]], axis=2).reshape(H, 2 * W, C)
    row_odd = jnp.stack([taps[2], taps[3]], axis=2).reshape(H, 2 * W, C)
    up = jnp.stack([row_even, row_odd], axis=1).reshape(2 * H, 2 * W, C)

    skip_hwc = jnp.transpose(
        skip_ref[0].reshape(C, 2 * H * 2 * W).astype(jnp.bfloat16),
        (1, 0)).reshape(2 * H, 2 * W, C)
    cat = jnp.concatenate([up, skip_hwc], axis=-1)
    zrow = jnp.zeros((1, 2 * W, 2 * C), jnp.bfloat16)
    zcol = jnp.zeros((2 * H + 2, 1, 2 * C), jnp.bfloat16)
    xs = jnp.concatenate([zrow, cat, zrow], axis=0)
    xs = jnp.concatenate([zcol, xs, zcol], axis=1)

    M = 2 * H * 2 * W
    acc = jnp.zeros((M, C), jnp.float32)
    for kh in range(3):
        rows = xs[kh:kh + 2 * H]
        a = jnp.concatenate(
            [rows[:, 0:2 * W], rows[:, 1:2 * W + 1], rows[:, 2:2 * W + 2]],
            axis=-1).reshape(M, 3 * 2 * C)
        acc = acc + jnp.dot(a, w1_ref[kh], preferred_element_type=jnp.float32)
    h = jnp.maximum(acc + s1_ref[...], 0.0).astype(jnp.bfloat16)
    h = h.reshape(2 * H, 2 * W, C)

    zrow = jnp.zeros((1, 2 * W, C), jnp.bfloat16)
    zcol = jnp.zeros((2 * H + 2, 1, C), jnp.bfloat16)
    hs = jnp.concatenate([zrow, h, zrow], axis=0)
    hs = jnp.concatenate([zcol, hs, zcol], axis=1)
    acc2 = jnp.zeros((M, C), jnp.float32)
    for kh in range(3):
        rows = hs[kh:kh + 2 * H]
        a = jnp.concatenate(
            [rows[:, 0:2 * W], rows[:, 1:2 * W + 1], rows[:, 2:2 * W + 2]],
            axis=-1).reshape(M, 3 * C)
        acc2 = acc2 + jnp.dot(a, w2_ref[kh], preferred_element_type=jnp.float32)
    y = jnp.maximum(acc2 + s2_ref[...], 0.0)
    o_ref[...] = jnp.transpose(y, (1, 0)).reshape(1, C, 2 * H, 2 * W)


def kernel(x_nchw, skip_nchw, up_w, up_b, c1_w, bn1_g, bn1_b, bn1_m, bn1_v,
           c2_w, bn2_g, bn2_b, bn2_m, bn2_v, *, eps=1e-5):
    N, Cin, H, W = x_nchw.shape
    C = up_w.shape[1]

    wup = jnp.transpose(up_w, (2, 3, 0, 1)).reshape(4, Cin, C)
    wup = wup.astype(jnp.bfloat16)
    bup = up_b.reshape(1, C)

    inv1 = bn1_g / jnp.sqrt(bn1_v + eps)
    inv2 = bn2_g / jnp.sqrt(bn2_v + eps)
    w1 = jnp.transpose(c1_w * inv1[:, None, None, None], (2, 3, 1, 0))
    w1 = w1.reshape(3, 3 * c1_w.shape[1], C).astype(jnp.bfloat16)
    s1 = (bn1_b - bn1_m * inv1).reshape(1, C)
    w2 = jnp.transpose(c2_w * inv2[:, None, None, None], (2, 3, 1, 0))
    w2 = w2.reshape(3, 3 * c2_w.shape[1], C).astype(jnp.bfloat16)
    s2 = (bn2_b - bn2_m * inv2).reshape(1, C)

    out = pl.pallas_call(
        _fused_decoder_kernel,
        out_shape=jax.ShapeDtypeStruct((N, C, 2 * H, 2 * W), jnp.float32),
        grid=(N,),
        in_specs=[
            pl.BlockSpec((1, Cin, H, W), lambda n: (n, 0, 0, 0)),
            pl.BlockSpec((1, C, 2 * H, 2 * W), lambda n: (n, 0, 0, 0)),
            pl.BlockSpec(wup.shape, lambda n: (0, 0, 0)),
            pl.BlockSpec(bup.shape, lambda n: (0, 0)),
            pl.BlockSpec(w1.shape, lambda n: (0, 0, 0)),
            pl.BlockSpec(s1.shape, lambda n: (0, 0)),
            pl.BlockSpec(w2.shape, lambda n: (0, 0, 0)),
            pl.BlockSpec(s2.shape, lambda n: (0, 0)),
        ],
        out_specs=pl.BlockSpec((1, C, 2 * H, 2 * W), lambda n: (n, 0, 0, 0)),
        compiler_params=pltpu.CompilerParams(
            dimension_semantics=("parallel",),
            vmem_limit_bytes=_VMEM_LIMIT,
        ),
    )(x_nchw, skip_nchw, wup, bup, w1, s1, w2, s2)

    return out


# speedup vs baseline: 1.5569x; 1.5569x over previous
import jax
import jax.numpy as jnp
from jax.experimental import pallas as pl
from jax.experimental.pallas import tpu as pltpu

_VMEM_LIMIT = 96 * 1024 * 1024


def _fused_decoder_kernel(x_ref, skip_ref, wup_ref, bup_ref, w1_ref, s1_ref,
                          w2_ref, s2_ref, o_ref):
    H, W, Cin = 32, 32, 256
    C = 128

    xc = x_ref[0].astype(jnp.bfloat16)
    b = bup_ref[...]
    taps = []
    for k in range(4):
        r = jax.lax.dot_general(
            xc, wup_ref[k], (((0,), (0,)), ((), ())),
            preferred_element_type=jnp.float32) + b
        taps.append(r.reshape(H, W, C).astype(jnp.bfloat16))
    row_even = jnp.stack([taps[0], taps[1]], axis=2).reshape(H, 2 * W, C)
    row_odd = jnp.stack([taps[2], taps[3]], axis=2).reshape(H, 2 * W, C)
    up = jnp.stack([row_even, row_odd], axis=1).reshape(2 * H, 2 * W, C)

    skip_hwc = jnp.transpose(
        skip_ref[0].astype(jnp.bfloat16), (1, 0)).reshape(2 * H, 2 * W, C)
    cat = jnp.concatenate([up, skip_hwc], axis=-1)
    zrow = jnp.zeros((1, 2 * W, 2 * C), jnp.bfloat16)
    zcol = jnp.zeros((2 * H + 2, 1, 2 * C), jnp.bfloat16)
    xs = jnp.concatenate([zrow, cat, zrow], axis=0)
    xs = jnp.concatenate([zcol, xs, zcol], axis=1)

    M = 2 * H * 2 * W
    acc = jnp.zeros((M, C), jnp.float32)
    for kh in range(3):
        rows = xs[kh:kh + 2 * H]
        a = jnp.concatenate(
            [rows[:, 0:2 * W], rows[:, 1:2 * W + 1], rows[:, 2:2 * W + 2]],
            axis=-1).reshape(M, 3 * 2 * C)
        acc = acc + jnp.dot(a, w1_ref[kh], preferred_element_type=jnp.float32)
    h = jnp.maximum(acc + s1_ref[...], 0.0).astype(jnp.bfloat16)
    h = h.reshape(2 * H, 2 * W, C)

    zrow = jnp.zeros((1, 2 * W, C), jnp.bfloat16)
    zcol = jnp.zeros((2 * H + 2, 1, C), jnp.bfloat16)
    hs = jnp.concatenate([zrow, h, zrow], axis=0)
    hs = jnp.concatenate([zcol, hs, zcol], axis=1)
    acc2 = jnp.zeros((M, C), jnp.float32)
    for kh in range(3):
        rows = hs[kh:kh + 2 * H]
        a = jnp.concatenate(
            [rows[:, 0:2 * W], rows[:, 1:2 * W + 1], rows[:, 2:2 * W + 2]],
            axis=-1).reshape(M, 3 * C)
        acc2 = acc2 + jnp.dot(a, w2_ref[kh], preferred_element_type=jnp.float32)
    y = jnp.maximum(acc2 + s2_ref[...], 0.0)
    o_ref[...] = jnp.transpose(y, (1, 0)).reshape(1, C, 2 * H * 2 * W)


def kernel(x_nchw, skip_nchw, up_w, up_b, c1_w, bn1_g, bn1_b, bn1_m, bn1_v,
           c2_w, bn2_g, bn2_b, bn2_m, bn2_v, *, eps=1e-5):
    N, Cin, H, W = x_nchw.shape
    C = up_w.shape[1]

    wup = jnp.transpose(up_w, (2, 3, 0, 1)).reshape(4, Cin, C)
    wup = wup.astype(jnp.bfloat16)
    bup = up_b.reshape(1, C)

    inv1 = bn1_g / jnp.sqrt(bn1_v + eps)
    inv2 = bn2_g / jnp.sqrt(bn2_v + eps)
    w1 = jnp.transpose(c1_w * inv1[:, None, None, None], (2, 3, 1, 0))
    w1 = w1.reshape(3, 3 * c1_w.shape[1], C).astype(jnp.bfloat16)
    s1 = (bn1_b - bn1_m * inv1).reshape(1, C)
    w2 = jnp.transpose(c2_w * inv2[:, None, None, None], (2, 3, 1, 0))
    w2 = w2.reshape(3, 3 * c2_w.shape[1], C).astype(jnp.bfloat16)
    s2 = (bn2_b - bn2_m * inv2).reshape(1, C)

    x3 = x_nchw.reshape(N, Cin, H * W)
    skip3 = skip_nchw.reshape(N, C, 2 * H * 2 * W)

    out = pl.pallas_call(
        _fused_decoder_kernel,
        out_shape=jax.ShapeDtypeStruct((N, C, 2 * H * 2 * W), jnp.float32),
        grid=(N,),
        in_specs=[
            pl.BlockSpec((1, Cin, H * W), lambda n: (n, 0, 0)),
            pl.BlockSpec((1, C, 2 * H * 2 * W), lambda n: (n, 0, 0)),
            pl.BlockSpec(wup.shape, lambda n: (0, 0, 0)),
            pl.BlockSpec(bup.shape, lambda n: (0, 0)),
            pl.BlockSpec(w1.shape, lambda n: (0, 0, 0)),
            pl.BlockSpec(s1.shape, lambda n: (0, 0)),
            pl.BlockSpec(w2.shape, lambda n: (0, 0, 0)),
            pl.BlockSpec(s2.shape, lambda n: (0, 0)),
        ],
        out_specs=pl.BlockSpec((1, C, 2 * H * 2 * W), lambda n: (n, 0, 0)),
        compiler_params=pltpu.CompilerParams(
            dimension_semantics=("parallel",),
            vmem_limit_bytes=_VMEM_LIMIT,
        ),
    )(x3, skip3, wup, bup, w1, s1, w2, s2)

    return out.reshape(N, C, 2 * H, 2 * W)


# speedup vs baseline: 2.2800x; 1.4644x over previous
import jax
import jax.numpy as jnp
from jax.experimental import pallas as pl
from jax.experimental.pallas import tpu as pltpu

_VMEM_LIMIT = 96 * 1024 * 1024


def _fused_decoder_kernel(x_ref, skip_ref, wup_ref, bup_ref, w1_ref, s1_ref,
                          w2_ref, s2_ref, o_ref):
    H, W, Cin = 32, 32, 256
    C = 128

    x2d = x_ref[...].reshape(H * W, Cin).astype(jnp.bfloat16)
    b = bup_ref[...]
    taps = []
    for k in range(4):
        r = jnp.dot(x2d, wup_ref[k], preferred_element_type=jnp.float32) + b
        taps.append(r.reshape(H, W, C).astype(jnp.bfloat16))
    row_even = jnp.stack([taps[0], taps[1]], axis=2).reshape(H, 2 * W, C)
    row_odd = jnp.stack([taps[2], taps[3]], axis=2).reshape(H, 2 * W, C)
    up = jnp.stack([row_even, row_odd], axis=1).reshape(2 * H, 2 * W, C)

    skip_hwc = skip_ref[0].astype(jnp.bfloat16)
    cat = jnp.concatenate([up, skip_hwc], axis=-1)
    zrow = jnp.zeros((1, 2 * W, 2 * C), jnp.bfloat16)
    zcol = jnp.zeros((2 * H + 2, 1, 2 * C), jnp.bfloat16)
    xs = jnp.concatenate([zrow, cat, zrow], axis=0)
    xs = jnp.concatenate([zcol, xs, zcol], axis=1)

    M = 2 * H * 2 * W
    acc = jnp.zeros((M, C), jnp.float32)
    for kh in range(3):
        rows = xs[kh:kh + 2 * H]
        a = jnp.concatenate(
            [rows[:, 0:2 * W], rows[:, 1:2 * W + 1], rows[:, 2:2 * W + 2]],
            axis=-1).reshape(M, 3 * 2 * C)
        acc = acc + jnp.dot(a, w1_ref[kh], preferred_element_type=jnp.float32)
    h = jnp.maximum(acc + s1_ref[...], 0.0).astype(jnp.bfloat16)
    h = h.reshape(2 * H, 2 * W, C)

    zrow = jnp.zeros((1, 2 * W, C), jnp.bfloat16)
    zcol = jnp.zeros((2 * H + 2, 1, C), jnp.bfloat16)
    hs = jnp.concatenate([zrow, h, zrow], axis=0)
    hs = jnp.concatenate([zcol, hs, zcol], axis=1)
    acc2 = jnp.zeros((M, C), jnp.float32)
    for kh in range(3):
        rows = hs[kh:kh + 2 * H]
        a = jnp.concatenate(
            [rows[:, 0:2 * W], rows[:, 1:2 * W + 1], rows[:, 2:2 * W + 2]],
            axis=-1).reshape(M, 3 * C)
        acc2 = acc2 + jnp.dot(a, w2_ref[kh], preferred_element_type=jnp.float32)
    y = jnp.maximum(acc2 + s2_ref[...], 0.0)
    o_ref[...] = y.reshape(1, 2 * H, 2 * W, C)


def kernel(x_nchw, skip_nchw, up_w, up_b, c1_w, bn1_g, bn1_b, bn1_m, bn1_v,
           c2_w, bn2_g, bn2_b, bn2_m, bn2_v, *, eps=1e-5):
    N, Cin, H, W = x_nchw.shape
    C = up_w.shape[1]

    wup = jnp.transpose(up_w, (2, 3, 0, 1)).reshape(4, Cin, C)
    wup = wup.astype(jnp.bfloat16)
    bup = up_b.reshape(1, C)

    inv1 = bn1_g / jnp.sqrt(bn1_v + eps)
    inv2 = bn2_g / jnp.sqrt(bn2_v + eps)
    w1 = jnp.transpose(c1_w * inv1[:, None, None, None], (2, 3, 1, 0))
    w1 = w1.reshape(3, 3 * c1_w.shape[1], C).astype(jnp.bfloat16)
    s1 = (bn1_b - bn1_m * inv1).reshape(1, C)
    w2 = jnp.transpose(c2_w * inv2[:, None, None, None], (2, 3, 1, 0))
    w2 = w2.reshape(3, 3 * c2_w.shape[1], C).astype(jnp.bfloat16)
    s2 = (bn2_b - bn2_m * inv2).reshape(1, C)

    x = jnp.transpose(x_nchw, (0, 2, 3, 1))
    skip = jnp.transpose(skip_nchw, (0, 2, 3, 1))

    out = pl.pallas_call(
        _fused_decoder_kernel,
        out_shape=jax.ShapeDtypeStruct((N, 2 * H, 2 * W, C), jnp.float32),
        grid=(N,),
        in_specs=[
            pl.BlockSpec((1, H, W, Cin), lambda n: (n, 0, 0, 0)),
            pl.BlockSpec((1, 2 * H, 2 * W, C), lambda n: (n, 0, 0, 0)),
            pl.BlockSpec(wup.shape, lambda n: (0, 0, 0)),
            pl.BlockSpec(bup.shape, lambda n: (0, 0)),
            pl.BlockSpec(w1.shape, lambda n: (0, 0, 0)),
            pl.BlockSpec(s1.shape, lambda n: (0, 0)),
            pl.BlockSpec(w2.shape, lambda n: (0, 0, 0)),
            pl.BlockSpec(s2.shape, lambda n: (0, 0)),
        ],
        out_specs=pl.BlockSpec((1, 2 * H, 2 * W, C), lambda n: (n, 0, 0, 0)),
        compiler_params=pltpu.CompilerParams(
            dimension_semantics=("parallel",),
            vmem_limit_bytes=_VMEM_LIMIT,
        ),
    )(x, skip, wup, bup, w1, s1, w2, s2)

    return jnp.transpose(out, (0, 3, 1, 2))


# speedup vs baseline: 2.8888x; 1.2670x over previous
import jax
import jax.numpy as jnp
from jax.experimental import pallas as pl
from jax.experimental.pallas import tpu as pltpu

_VMEM_LIMIT = 64 * 1024 * 1024


def _fused_decoder_kernel(x_ref, skip_ref, wup_ref, bup_ref, w1_ref, s1_ref,
                          w2_ref, s2_ref, o_ref):
    H, W, Cin = 32, 32, 256
    C = 128
    M = 2 * H * 2 * W

    x2d = x_ref[...].reshape(H * W, Cin).astype(jnp.bfloat16)
    p_up = jnp.dot(x2d, wup_ref[...], preferred_element_type=jnp.float32)
    b = bup_ref[...]
    taps = [(p_up[:, k * C:(k + 1) * C] + b).astype(jnp.bfloat16)
            .reshape(H, W, C) for k in range(4)]
    row_even = jnp.stack([taps[0], taps[1]], axis=2).reshape(H, 2 * W, C)
    row_odd = jnp.stack([taps[2], taps[3]], axis=2).reshape(H, 2 * W, C)
    up = jnp.stack([row_even, row_odd], axis=1).reshape(M, C)

    wpos = jax.lax.broadcasted_iota(jnp.int32, (M, C), 0) % (2 * W)
    m_wfirst = wpos != 0
    m_wlast = wpos != 2 * W - 1
    zrow = jnp.zeros((1, C), jnp.float32)
    zblk = jnp.zeros((2 * W, C), jnp.float32)
    fz = jnp.float32(0.0)

    def combine_kw(qs):
        r0 = jnp.concatenate([zrow, qs[0][:-1]], axis=0)
        r2 = jnp.concatenate([qs[2][1:], zrow], axis=0)
        return (qs[1] + jnp.where(m_wfirst, r0, fz)
                + jnp.where(m_wlast, r2, fz))

    skip_bf = skip_ref[...].reshape(M, C).astype(jnp.bfloat16)
    x1 = jnp.concatenate([up, skip_bf], axis=1)
    qs = []
    for kw in range(3):
        p = jnp.dot(x1, w1_ref[kw], preferred_element_type=jnp.float32)
        q = p[:, C:2 * C]
        q = q + jnp.concatenate([zblk, p[:-2 * W, 0:C]], axis=0)
        q = q + jnp.concatenate([p[2 * W:, 2 * C:3 * C], zblk], axis=0)
        qs.append(q)
    h1 = combine_kw(qs)
    h1 = jnp.maximum(h1 + s1_ref[...], 0.0).astype(jnp.bfloat16)

    zblk_bf = jnp.zeros((2 * W, C), jnp.bfloat16)
    h_dn = jnp.concatenate([h1[2 * W:], zblk_bf], axis=0)
    x2 = jnp.concatenate([h1, h_dn], axis=1)
    qs = []
    for kw in range(3):
        p = jnp.dot(x2, w2_ref[kw], preferred_element_type=jnp.float32)
        q = p[:, 0:C]
        q = q + jnp.concatenate([zblk, p[:-2 * W, C:2 * C]], axis=0)
        qs.append(q)
    y = combine_kw(qs)
    y = jnp.maximum(y + s2_ref[...], 0.0)
    o_ref[...] = y.reshape(1, 2 * H, 2 * W, C)


def kernel(x_nchw, skip_nchw, up_w, up_b, c1_w, bn1_g, bn1_b, bn1_m, bn1_v,
           c2_w, bn2_g, bn2_b, bn2_m, bn2_v, *, eps=1e-5):
    N, Cin, H, W = x_nchw.shape
    C = up_w.shape[1]
    f32 = jnp.float32

    wup = jnp.transpose(up_w, (2, 3, 0, 1)).reshape(4, Cin, C)
    wup = jnp.concatenate([wup[k] for k in range(4)], axis=1)
    wup = wup.astype(jnp.bfloat16)
    bup = up_b.reshape(1, C)

    inv1 = bn1_g / jnp.sqrt(bn1_v + eps)
    inv2 = bn2_g / jnp.sqrt(bn2_v + eps)
    w1s = c1_w * inv1[:, None, None, None]
    w2s = c2_w * inv2[:, None, None, None]
    s1 = (bn1_b - bn1_m * inv1).reshape(1, C)
    s2 = (bn2_b - bn2_m * inv2).reshape(1, C)

    Cin1 = w1s.shape[1]
    w1 = jnp.stack([
        jnp.concatenate([w1s[:, :, kh, kw].T for kh in range(3)], axis=1)
        for kw in range(3)])
    w1 = w1.astype(jnp.bfloat16)

    zkk = jnp.zeros((C, C), f32)
    w2 = jnp.stack([
        jnp.concatenate([
            jnp.concatenate([w2s[:, :, 1, kw].T, w2s[:, :, 2, kw].T], axis=0),
            jnp.concatenate([w2s[:, :, 0, kw].T, zkk], axis=0),
        ], axis=1)
        for kw in range(3)])
    w2 = w2.astype(jnp.bfloat16)

    x = jnp.transpose(x_nchw, (0, 2, 3, 1))
    skip = jnp.transpose(skip_nchw, (0, 2, 3, 1))

    out = pl.pallas_call(
        _fused_decoder_kernel,
        out_shape=jax.ShapeDtypeStruct((N, 2 * H, 2 * W, C), jnp.float32),
        grid=(N,),
        in_specs=[
            pl.BlockSpec((1, H, W, Cin), lambda n: (n, 0, 0, 0)),
            pl.BlockSpec((1, 2 * H, 2 * W, C), lambda n: (n, 0, 0, 0)),
            pl.BlockSpec(wup.shape, lambda n: (0, 0)),
            pl.BlockSpec(bup.shape, lambda n: (0, 0)),
            pl.BlockSpec(w1.shape, lambda n: (0, 0, 0)),
            pl.BlockSpec(s1.shape, lambda n: (0, 0)),
            pl.BlockSpec(w2.shape, lambda n: (0, 0, 0)),
            pl.BlockSpec(s2.shape, lambda n: (0, 0)),
        ],
        out_specs=pl.BlockSpec((1, 2 * H, 2 * W, C), lambda n: (n, 0, 0, 0)),
        compiler_params=pltpu.CompilerParams(
            dimension_semantics=("parallel",),
            vmem_limit_bytes=_VMEM_LIMIT,
        ),
    )(x, skip, wup, bup, w1, s1, w2, s2)

    return jnp.transpose(out, (0, 3, 1, 2))


# speedup vs baseline: 3.2436x; 1.1228x over previous
import jax
import jax.numpy as jnp
from jax.experimental import pallas as pl
from jax.experimental.pallas import tpu as pltpu

_VMEM_LIMIT = 64 * 1024 * 1024


_IMGS = 2


def _fused_decoder_kernel(x_ref, skip_ref, wup_ref, bup_ref, w1_ref, s1_ref,
                          w2_ref, s2_ref, o_ref):
    H, W, Cin = 32, 32, 256
    C = 128
    M = 2 * H * 2 * W

    Wo = 2 * W
    zcol = jnp.zeros((2 * H, 1, C), jnp.float32)
    zblk_bf = jnp.zeros((Wo, C), jnp.bfloat16)
    b = bup_ref[...]

    def combine_kw(qs):
        q0 = qs[0].reshape(2 * H, Wo, C)
        q2 = qs[2].reshape(2 * H, Wo, C)
        r0 = jnp.concatenate([zcol, q0[:, :-1]], axis=1).reshape(M, C)
        r2 = jnp.concatenate([q2[:, 1:], zcol], axis=1).reshape(M, C)
        return qs[1] + r0 + r2

    def shifted_sum3(p0, p1, p2):
        top = p1[0:Wo] + p2[Wo:2 * Wo]
        mid = p1[Wo:M - Wo] + p2[2 * Wo:] + p0[:M - 2 * Wo]
        bot = p1[M - Wo:] + p0[M - 2 * Wo:M - Wo]
        return jnp.concatenate([top, mid, bot], axis=0)

    def one_image(j):
        x2d = x_ref[j].reshape(H * W, Cin).astype(jnp.bfloat16)
        p_up = jnp.dot(x2d, wup_ref[...], preferred_element_type=jnp.float32)
        taps = [(p_up[:, k * C:(k + 1) * C] + b).reshape(H, W, C)
                for k in range(4)]
        row_even = jnp.stack([taps[0], taps[1]], axis=2).reshape(H, Wo, C)
        row_odd = jnp.stack([taps[2], taps[3]], axis=2).reshape(H, Wo, C)
        up = jnp.stack([row_even, row_odd], axis=1).reshape(M, C)
        up = up.astype(jnp.bfloat16)

        skip_bf = skip_ref[j].reshape(M, C).astype(jnp.bfloat16)
        x1 = jnp.concatenate([up, skip_bf], axis=1)
        qs = []
        for kw in range(3):
            p = jnp.dot(x1, w1_ref[kw], preferred_element_type=jnp.float32)
            qs.append(shifted_sum3(p[:, 0:C], p[:, C:2 * C], p[:, 2 * C:]))
        h1 = combine_kw(qs)
        h1 = jnp.maximum(h1 + s1_ref[...], 0.0).astype(jnp.bfloat16)

        h_dn = jnp.concatenate([h1[Wo:], zblk_bf], axis=0)
        x2 = jnp.concatenate([h1, h_dn], axis=1)
        qs = []
        for kw in range(3):
            p = jnp.dot(x2, w2_ref[kw], preferred_element_type=jnp.float32)
            p1 = p[:, 0:C]
            p0 = p[:, C:2 * C]
            q = jnp.concatenate(
                [p1[0:Wo], p1[Wo:] + p0[:M - Wo]], axis=0)
            qs.append(q)
        y = combine_kw(qs)
        y = jnp.maximum(y + s2_ref[...], 0.0)
        o_ref[j] = y.reshape(2 * H, Wo, C)

    for j in range(_IMGS):
        one_image(j)


def kernel(x_nchw, skip_nchw, up_w, up_b, c1_w, bn1_g, bn1_b, bn1_m, bn1_v,
           c2_w, bn2_g, bn2_b, bn2_m, bn2_v, *, eps=1e-5):
    N, Cin, H, W = x_nchw.shape
    C = up_w.shape[1]
    f32 = jnp.float32

    wup = jnp.transpose(up_w, (2, 3, 0, 1)).reshape(4, Cin, C)
    wup = jnp.concatenate([wup[k] for k in range(4)], axis=1)
    wup = wup.astype(jnp.bfloat16)
    bup = up_b.reshape(1, C)

    inv1 = bn1_g / jnp.sqrt(bn1_v + eps)
    inv2 = bn2_g / jnp.sqrt(bn2_v + eps)
    w1s = c1_w * inv1[:, None, None, None]
    w2s = c2_w * inv2[:, None, None, None]
    s1 = (bn1_b - bn1_m * inv1).reshape(1, C)
    s2 = (bn2_b - bn2_m * inv2).reshape(1, C)

    w1 = jnp.stack([
        jnp.concatenate([w1s[:, :, kh, kw].T for kh in range(3)], axis=1)
        for kw in range(3)])
    w1 = w1.astype(jnp.bfloat16)

    zkk = jnp.zeros((C, C), f32)
    w2 = jnp.stack([
        jnp.concatenate([
            jnp.concatenate([w2s[:, :, 1, kw].T, w2s[:, :, 2, kw].T], axis=0),
            jnp.concatenate([w2s[:, :, 0, kw].T, zkk], axis=0),
        ], axis=1)
        for kw in range(3)])
    w2 = w2.astype(jnp.bfloat16)

    x = jnp.transpose(x_nchw, (0, 2, 3, 1))
    skip = jnp.transpose(skip_nchw, (0, 2, 3, 1))

    out = pl.pallas_call(
        _fused_decoder_kernel,
        out_shape=jax.ShapeDtypeStruct((N, 2 * H, 2 * W, C), jnp.float32),
        grid=(N // _IMGS,),
        in_specs=[
            pl.BlockSpec((_IMGS, H, W, Cin), lambda n: (n, 0, 0, 0)),
            pl.BlockSpec((_IMGS, 2 * H, 2 * W, C), lambda n: (n, 0, 0, 0)),
            pl.BlockSpec(wup.shape, lambda n: (0, 0)),
            pl.BlockSpec(bup.shape, lambda n: (0, 0)),
            pl.BlockSpec(w1.shape, lambda n: (0, 0, 0)),
            pl.BlockSpec(s1.shape, lambda n: (0, 0)),
            pl.BlockSpec(w2.shape, lambda n: (0, 0, 0)),
            pl.BlockSpec(s2.shape, lambda n: (0, 0)),
        ],
        out_specs=pl.BlockSpec((_IMGS, 2 * H, 2 * W, C), lambda n: (n, 0, 0, 0)),
        compiler_params=pltpu.CompilerParams(
            dimension_semantics=("parallel",),
            vmem_limit_bytes=_VMEM_LIMIT,
        ),
    )(x, skip, wup, bup, w1, s1, w2, s2)

    return jnp.transpose(out, (0, 3, 1, 2))


# speedup vs baseline: 3.4984x; 1.0786x over previous
import jax
import jax.numpy as jnp
from jax.experimental import pallas as pl
from jax.experimental.pallas import tpu as pltpu

_VMEM_LIMIT = 64 * 1024 * 1024


_IMGS = 2


def _fused_decoder_kernel(x_ref, skip_ref, wup_ref, bup_ref, w1_ref, s1_ref,
                          w2_ref, s2_ref, o_ref):
    H, W, Cin = 32, 32, 256
    C = 128
    M = 2 * H * 2 * W

    Wo = 2 * W
    zcol = jnp.zeros((2 * H, 1, C), jnp.float32)
    zblk_bf = jnp.zeros((Wo, C), jnp.bfloat16)
    b = bup_ref[...]

    def combine_kw(qs):
        q0 = qs[0].reshape(2 * H, Wo, C)
        q2 = qs[2].reshape(2 * H, Wo, C)
        r0 = jnp.concatenate([zcol, q0[:, :-1]], axis=1).reshape(M, C)
        r2 = jnp.concatenate([q2[:, 1:], zcol], axis=1).reshape(M, C)
        return qs[1] + r0 + r2

    def shifted_sum3(p0, p1, p2):
        top = p1[0:Wo] + p2[Wo:2 * Wo]
        mid = p1[Wo:M - Wo] + p2[2 * Wo:] + p0[:M - 2 * Wo]
        bot = p1[M - Wo:] + p0[M - 2 * Wo:M - Wo]
        return jnp.concatenate([top, mid, bot], axis=0)

    def deconv(j):
        x2d = x_ref[j].reshape(H * W, Cin).astype(jnp.bfloat16)
        p_up = jnp.dot(x2d, wup_ref[...], preferred_element_type=jnp.float32)
        taps = [(p_up[:, k * C:(k + 1) * C] + b).reshape(H, W, C)
                for k in range(4)]
        row_even = jnp.stack([taps[0], taps[1]], axis=2).reshape(H, Wo, C)
        row_odd = jnp.stack([taps[2], taps[3]], axis=2).reshape(H, Wo, C)
        up = jnp.stack([row_even, row_odd], axis=1).reshape(M, C)
        return up.astype(jnp.bfloat16)

    def conv1(j, up):
        skip_bf = skip_ref[j].reshape(M, C).astype(jnp.bfloat16)
        x1 = jnp.concatenate([up, skip_bf], axis=1)
        p = jnp.dot(x1, w1_ref[...], preferred_element_type=jnp.float32)
        qs = [shifted_sum3(p[:, kw * 3 * C:kw * 3 * C + C],
                           p[:, kw * 3 * C + C:kw * 3 * C + 2 * C],
                           p[:, kw * 3 * C + 2 * C:(kw + 1) * 3 * C])
              for kw in range(3)]
        h1 = combine_kw(qs)
        return jnp.maximum(h1 + s1_ref[...], 0.0).astype(jnp.bfloat16)

    def conv2(j, h1):
        h_dn = jnp.concatenate([h1[Wo:], zblk_bf], axis=0)
        x2 = jnp.concatenate([h1, h_dn], axis=1)
        p = jnp.dot(x2, w2_ref[...], preferred_element_type=jnp.float32)
        qs = []
        for kw in range(3):
            p1 = p[:, kw * 2 * C:kw * 2 * C + C]
            p0 = p[:, kw * 2 * C + C:(kw + 1) * 2 * C]
            q = jnp.concatenate(
                [p1[0:Wo], p1[Wo:] + p0[:M - Wo]], axis=0)
            qs.append(q)
        y = combine_kw(qs)
        y = jnp.maximum(y + s2_ref[...], 0.0)
        o_ref[j] = y.reshape(2 * H, Wo, C)

    ups = [deconv(j) for j in range(_IMGS)]
    h1s = [conv1(j, ups[j]) for j in range(_IMGS)]
    for j in range(_IMGS):
        conv2(j, h1s[j])


def kernel(x_nchw, skip_nchw, up_w, up_b, c1_w, bn1_g, bn1_b, bn1_m, bn1_v,
           c2_w, bn2_g, bn2_b, bn2_m, bn2_v, *, eps=1e-5):
    N, Cin, H, W = x_nchw.shape
    C = up_w.shape[1]
    f32 = jnp.float32

    wup = jnp.transpose(up_w, (2, 3, 0, 1)).reshape(4, Cin, C)
    wup = jnp.concatenate([wup[k] for k in range(4)], axis=1)
    wup = wup.astype(jnp.bfloat16)
    bup = up_b.reshape(1, C)

    inv1 = bn1_g / jnp.sqrt(bn1_v + eps)
    inv2 = bn2_g / jnp.sqrt(bn2_v + eps)
    w1s = c1_w * inv1[:, None, None, None]
    w2s = c2_w * inv2[:, None, None, None]
    s1 = (bn1_b - bn1_m * inv1).reshape(1, C)
    s2 = (bn2_b - bn2_m * inv2).reshape(1, C)

    w1 = jnp.concatenate([
        jnp.concatenate([w1s[:, :, kh, kw].T for kh in range(3)], axis=1)
        for kw in range(3)], axis=1)
    w1 = w1.astype(jnp.bfloat16)

    zkk = jnp.zeros((C, C), f32)
    w2 = jnp.concatenate([
        jnp.concatenate([
            jnp.concatenate([w2s[:, :, 1, kw].T, w2s[:, :, 2, kw].T], axis=0),
            jnp.concatenate([w2s[:, :, 0, kw].T, zkk], axis=0),
        ], axis=1)
        for kw in range(3)], axis=1)
    w2 = w2.astype(jnp.bfloat16)

    x = jnp.transpose(x_nchw, (0, 2, 3, 1))
    skip = jnp.transpose(skip_nchw, (0, 2, 3, 1))

    out = pl.pallas_call(
        _fused_decoder_kernel,
        out_shape=jax.ShapeDtypeStruct((N, 2 * H, 2 * W, C), jnp.float32),
        grid=(N // _IMGS,),
        in_specs=[
            pl.BlockSpec((_IMGS, H, W, Cin), lambda n: (n, 0, 0, 0)),
            pl.BlockSpec((_IMGS, 2 * H, 2 * W, C), lambda n: (n, 0, 0, 0)),
            pl.BlockSpec(wup.shape, lambda n: (0, 0)),
            pl.BlockSpec(bup.shape, lambda n: (0, 0)),
            pl.BlockSpec(w1.shape, lambda n: (0, 0)),
            pl.BlockSpec(s1.shape, lambda n: (0, 0)),
            pl.BlockSpec(w2.shape, lambda n: (0, 0)),
            pl.BlockSpec(s2.shape, lambda n: (0, 0)),
        ],
        out_specs=pl.BlockSpec((_IMGS, 2 * H, 2 * W, C), lambda n: (n, 0, 0, 0)),
        compiler_params=pltpu.CompilerParams(
            dimension_semantics=("parallel",),
            vmem_limit_bytes=_VMEM_LIMIT,
        ),
    )(x, skip, wup, bup, w1, s1, w2, s2)

    return jnp.transpose(out, (0, 3, 1, 2))
